# baseline (speedup 1.0000x reference)
import functools

import jax
import jax.numpy as jnp
from jax.experimental import pallas as pl
from jax.experimental.pallas import tpu as pltpu


def _accum(acc_ref, part, j):
    @pl.when(j == 0)
    def _():
        acc_ref[...] = part

    @pl.when(j != 0)
    def _():
        acc_ref[...] = acc_ref[...] + part


def _cgn_body(xt_ref, a_ref, u0t_ref, b0_ref, us_ref, out_ref,
              ht_scr, pack_scr, acc_ref, deg_ref, degall_ref,
              *, n, bi, bj, h1_tail):
    l = pl.program_id(0)
    i = pl.program_id(1)
    j = pl.program_id(2)
    nj = pl.num_programs(2)
    dn = (((1,), (0,)), ((), ()))

    @pl.when((l == 0) & (i == 0))
    def _():
        xt_b = xt_ref[:, pl.ds(j * bj, bj)]
        h0t = jax.lax.dot_general(u0t_ref[...], xt_b, dn,
                                  preferred_element_type=jnp.float32)
        h0t = h0t + b0_ref[...]
        col = jax.lax.broadcasted_iota(jnp.int32, h0t.shape, 1)
        h0t = jnp.where(col < n - j * bj, h0t, 0.0)
        ht_scr[0, :, pl.ds(j * bj, bj)] = h0t.astype(jnp.bfloat16)

    ht_b = ht_scr[l, :, pl.ds(j * bj, bj)]
    g = bj // 256

    @pl.when(l == 0)
    def _():
        a_s32 = a_ref[...]
        a_blk = a_s32.astype(jnp.bfloat16)
        part = jax.lax.dot_general(ht_b, a_blk, dn,
                                   preferred_element_type=jnp.float32)
        _accum(acc_ref, part, j)

        a4 = a_s32.reshape(g, 32, 8, bi)
        packed = a4[:, 0]
        for k in range(1, 32):
            packed = packed | (a4[:, k] << k)
        pack_scr[i, j] = packed

        row = jax.lax.broadcasted_iota(jnp.int32, (1, bj), 1)
        valid = (row < n - j * bj).astype(jnp.bfloat16)
        degp = jax.lax.dot_general(valid, a_blk, dn,
                                   preferred_element_type=jnp.float32)
        _accum(deg_ref, degp, j)

        @pl.when(j == nj - 1)
        def _():
            degall_ref[:, pl.ds(i * bi, bi)] = deg_ref[...]

    @pl.when(l == 1)
    def _():
        packed = pack_scr[i, j]
        bits = [((packed >> k) & 1) for k in range(32)]
        a_blk = jnp.stack(bits, axis=1).reshape(bj, bi).astype(jnp.bfloat16)
        part = jax.lax.dot_general(ht_b, a_blk, dn,
                                   preferred_element_type=jnp.float32)
        _accum(acc_ref, part, j)

    @pl.when(j == nj - 1)
    def _():
        deg_row = degall_ref[:, pl.ds(i * bi, bi)]
        scaled = (acc_ref[...] / deg_row).astype(jnp.bfloat16)
        y = jax.lax.dot_general(us_ref[0], scaled, dn,
                                preferred_element_type=jnp.float32)
        y = jnp.maximum(y, 0.0)

        @pl.when(l == 0)
        def _():
            col = jax.lax.broadcasted_iota(jnp.int32, y.shape, 1)
            y0 = jnp.where(col < n - i * bi, y, 0.0)
            ht_scr[1, :, pl.ds(i * bi, bi)] = y0.astype(jnp.bfloat16)
            if h1_tail:
                ni_ = pl.num_programs(1)

                @pl.when(i == ni_ - 1)
                def _():
                    d_ = y.shape[0]
                    ht_scr[1, :, pl.ds(ni_ * bi, h1_tail)] = jnp.zeros(
                        (d_, h1_tail), jnp.bfloat16)

        @pl.when(l == 1)
        def _():
            out_ref[...] = jnp.swapaxes(y, 0, 1)

def kernel(x, adj_mat, U0, b0, U1, U2):
    n, d = x.shape
    bi = 1280 if n > 2560 else 256
    bj = 1280 if n > 2560 else 512
    ni = -(-n // bi)
    nj = -(-n // bj)
    n_pad = max(ni * bi, nj * bj)

    xt = jnp.zeros((d, n_pad), jnp.bfloat16).at[:, :n].set(
        x.astype(jnp.bfloat16).T)
    u0t = U0.T.astype(jnp.bfloat16)
    us = jnp.stack([U1, U2]).astype(jnp.bfloat16)
    b0c = b0.reshape(d, 1)

    body = functools.partial(_cgn_body, n=n, bi=bi, bj=bj,
                             h1_tail=n_pad - ni * bi)

    def a_idx(l, i, j):
        return (jnp.where(l == 0, j, 0), jnp.where(l == 0, i, 0))

    return pl.pallas_call(
        body,
        grid=(2, ni, nj),
        in_specs=[
            pl.BlockSpec((d, n_pad), lambda l, i, j: (0, 0)),
            pl.BlockSpec((bj, bi), a_idx),
            pl.BlockSpec((d, d), lambda l, i, j: (0, 0)),
            pl.BlockSpec((d, 1), lambda l, i, j: (0, 0)),
            pl.BlockSpec((1, d, d), lambda l, i, j: (l, 0, 0)),
        ],
        out_specs=pl.BlockSpec((bi, d), lambda l, i, j: (i, 0)),
        out_shape=jax.ShapeDtypeStruct((n, d), jnp.float32),
        scratch_shapes=[
            pltpu.VMEM((2, d, n_pad), jnp.bfloat16),
            pltpu.VMEM((ni, nj, bj // 256, 8, bi), jnp.int32),
            pltpu.VMEM((d, bi), jnp.float32),
            pltpu.VMEM((1, bi), jnp.float32),
            pltpu.VMEM((1, n_pad), jnp.float32),
        ],
    )(xt, adj_mat, u0t, b0c, us)


# speedup vs baseline: 1.2137x; 1.2137x over previous
import functools

import jax
import jax.numpy as jnp
from jax.experimental import pallas as pl
from jax.experimental.pallas import tpu as pltpu

_GRAN = 256
_NGRAN = 5
_NPACK = 3


def _accum(acc_ref, part, j):
    @pl.when(j == 0)
    def _():
        acc_ref[...] = part

    @pl.when(j != 0)
    def _():
        acc_ref[...] = acc_ref[...] + part


def _pack_bits(gran):
    packed = gran[0:8, :]
    for k in range(1, 32):
        packed = packed | (gran[8 * k:8 * k + 8, :] << k)
    return packed


def _unpack_bits(packed):
    return jnp.concatenate([(packed >> k) & 1 for k in range(32)], axis=0)


def _h0_body(xt_ref, u0t_ref, b0_ref, h0t_ref, *, n, bj):
    j = pl.program_id(0)
    h0t = jax.lax.dot_general(u0t_ref[...], xt_ref[...],
                              (((1,), (0,)), ((), ())),
                              preferred_element_type=jnp.float32)
    h0t = h0t + b0_ref[...]
    col = jax.lax.broadcasted_iota(jnp.int32, h0t.shape, 1)
    h0t_ref[...] = jnp.where(col < n - j * bj, h0t, 0.0).astype(jnp.bfloat16)


def _cgn_body(h0t_ref, *refs, n, bi, bj, h1_tail):
    (a0, a1, a2, a3, a4, us_ref, out_ref,
     ht_scr, pack_scr, acc_ref, deg_ref, degall_ref) = refs
    a_refs = (a0, a1, a2, a3, a4)
    l = pl.program_id(0)
    i = pl.program_id(1)
    j = pl.program_id(2)
    nj = pl.num_programs(2)
    dn = (((1,), (0,)), ((), ()))

    def matmul_accum(ht_b, a_blk):
        part = jax.lax.dot_general(ht_b, a_blk, dn,
                                   preferred_element_type=jnp.float32)
        _accum(acc_ref, part, j)
        return a_blk

    @pl.when(l == 0)
    def _():
        grans = [r[...] for r in a_refs]
        a_blk = matmul_accum(
            h0t_ref[:, pl.ds(j * bj, bj)],
            jnp.concatenate([g.astype(jnp.bfloat16) for g in grans], axis=0))
        for t in range(_NPACK):
            pack_scr[i, j, t] = _pack_bits(grans[t])

        row = jax.lax.broadcasted_iota(jnp.int32, (1, bj), 1)
        valid = (row < n - j * bj).astype(jnp.bfloat16)
        degp = jax.lax.dot_general(valid, a_blk, dn,
                                   preferred_element_type=jnp.float32)
        _accum(deg_ref, degp, j)

        @pl.when(j == nj - 1)
        def _():
            degall_ref[:, pl.ds(i * bi, bi)] = deg_ref[...]

    @pl.when(l == 1)
    def _():
        parts = [_unpack_bits(pack_scr[i, j, t]).astype(jnp.bfloat16)
                 for t in range(_NPACK)]
        parts += [a_refs[t][...].astype(jnp.bfloat16)
                  for t in range(_NPACK, _NGRAN)]
        matmul_accum(ht_scr[:, pl.ds(j * bj, bj)],
                     jnp.concatenate(parts, axis=0))

    @pl.when(j == nj - 1)
    def _():
        deg_row = degall_ref[:, pl.ds(i * bi, bi)]
        scaled = (acc_ref[...] / deg_row).astype(jnp.bfloat16)
        y = jax.lax.dot_general(us_ref[0], scaled, dn,
                                preferred_element_type=jnp.float32)
        y = jnp.maximum(y, 0.0)

        @pl.when(l == 0)
        def _():
            col = jax.lax.broadcasted_iota(jnp.int32, y.shape, 1)
            y0 = jnp.where(col < n - i * bi, y, 0.0)
            ht_scr[:, pl.ds(i * bi, bi)] = y0.astype(jnp.bfloat16)
            if h1_tail:
                ni_ = pl.num_programs(1)

                @pl.when(i == ni_ - 1)
                def _():
                    d_ = y.shape[0]
                    ht_scr[:, pl.ds(ni_ * bi, h1_tail)] = jnp.zeros(
                        (d_, h1_tail), jnp.bfloat16)

        @pl.when(l == 1)
        def _():
            out_ref[...] = jnp.swapaxes(y, 0, 1)

def kernel(x, adj_mat, U0, b0, U1, U2):
    n, d = x.shape
    bj = _GRAN * _NGRAN
    bi = 2560 if n > 2560 else 256
    ni = -(-n // bi)
    nj = -(-n // bj)
    n_pad = max(ni * bi, nj * bj)

    xt = jnp.zeros((d, n_pad), jnp.bfloat16).at[:, :n].set(
        x.astype(jnp.bfloat16).T)
    u0t = U0.T.astype(jnp.bfloat16)
    us = jnp.stack([U1, U2]).astype(jnp.bfloat16)
    b0c = b0.reshape(d, 1)

    h0t = pl.pallas_call(
        functools.partial(_h0_body, n=n, bj=bj),
        grid=(nj,),
        in_specs=[
            pl.BlockSpec((d, bj), lambda j: (0, j)),
            pl.BlockSpec((d, d), lambda j: (0, 0)),
            pl.BlockSpec((d, 1), lambda j: (0, 0)),
        ],
        out_specs=pl.BlockSpec((d, bj), lambda j: (0, j)),
        out_shape=jax.ShapeDtypeStruct((d, n_pad), jnp.bfloat16),
    )(xt, u0t, b0c)

    body = functools.partial(_cgn_body, n=n, bi=bi, bj=bj,
                             h1_tail=n_pad - ni * bi)

    def gran_idx(t):
        if t < _NPACK:
            return lambda l, i, j: (jnp.where(l == 0, _NGRAN * j + t, 0),
                                    jnp.where(l == 0, i, 0))
        return lambda l, i, j: (_NGRAN * j + t, i)

    gran_specs = [pl.BlockSpec((_GRAN, bi), gran_idx(t))
                  for t in range(_NGRAN)]

    return pl.pallas_call(
        body,
        grid=(2, ni, nj),
        in_specs=[
            pl.BlockSpec((d, n_pad), lambda l, i, j: (0, 0)),
            *gran_specs,
            pl.BlockSpec((1, d, d), lambda l, i, j: (l, 0, 0)),
        ],
        out_specs=pl.BlockSpec((bi, d), lambda l, i, j: (i, 0)),
        out_shape=jax.ShapeDtypeStruct((n, d), jnp.float32),
        scratch_shapes=[
            pltpu.VMEM((d, n_pad), jnp.bfloat16),
            pltpu.VMEM((ni, nj, _NPACK, 8, bi), jnp.int32),
            pltpu.VMEM((d, bi), jnp.float32),
            pltpu.VMEM((1, bi), jnp.float32),
            pltpu.VMEM((1, n_pad), jnp.float32),
        ],
    )(h0t, adj_mat, adj_mat, adj_mat, adj_mat, adj_mat, us)
